# baseline (speedup 1.0000x reference)
import functools

import jax
import jax.numpy as jnp
from jax import lax
from jax.experimental import pallas as pl
from jax.experimental.pallas import tpu as pltpu
from jax.experimental.pallas import tpu_sc as plsc

V = 1000000
TAIL0 = (V // 128) * 128
D = 64
B = 16384
NC = 2
NS = 16
NW = NC * NS
B_PER_W = B // NW
N_CHUNKS = B_PER_W // 16
DEPTH = 6
OUT_W = 16 * D

_mesh = plsc.VectorSubcoreMesh(core_axis_name="c", subcore_axis_name="s")


@functools.partial(
    pl.kernel,
    mesh=_mesh,
    out_type=jax.ShapeDtypeStruct((B * D,), jnp.float32),
    scratch_types=[
        pltpu.VMEM((B_PER_W,), jnp.int32),
        pltpu.VMEM((2 * OUT_W,), jnp.float32),
        pltpu.VMEM((D, 128), jnp.float32),
        *[pltpu.VMEM((D, 128), jnp.float32) for _ in range(DEPTH)],
        *[pltpu.SemaphoreType.DMA for _ in range(DEPTH)],
        pltpu.SemaphoreType.DMA,
        pltpu.SemaphoreType.DMA,
    ],
    compiler_params=pltpu.CompilerParams(needs_layout_passes=False),
)
def _gather_kernel(
    table_hbm, idx_hbm, tail_hbm, out_hbm, idx_v, outbuf, tail_buf, *rest
):
    slabs = rest[:DEPTH]
    sems = rest[DEPTH:2 * DEPTH]
    osems = rest[2 * DEPTH:2 * DEPTH + 2]
    wid = lax.axis_index("s") * NC + lax.axis_index("c")
    base = wid * B_PER_W
    pltpu.sync_copy(idx_hbm.at[pl.ds(base, B_PER_W)], idx_v)
    pltpu.sync_copy(tail_hbm, tail_buf)
    c_iota = lax.iota(jnp.int32, 16)

    def fire(slot, r):
        t128 = pl.multiple_of(
            jnp.where(r < TAIL0, (r >> 7) << 7, 0).astype(jnp.int32), 128
        )
        pltpu.async_copy(table_hbm.at[:, pl.ds(t128, 128)], slabs[slot], sems[slot])

    def wait_slab(slot):
        pltpu.make_async_copy(
            table_hbm.at[:, pl.ds(0, 128)], slabs[slot], sems[slot]
        ).wait()

    def wait_flush(half):
        pltpu.make_async_copy(
            outbuf.at[pl.ds(0, OUT_W)], out_hbm.at[pl.ds(0, OUT_W)], osems[half]
        ).wait()

    def chunk_body(c, half):
        vec = idx_v[pl.ds(c * 16, 16)]
        cnext = jnp.minimum(c + 1, N_CHUNKS - 1)
        vec_next = idx_v[pl.ds(cnext * 16, 16)]
        par = pl.multiple_of(half * OUT_W, 64)
        @pl.when(c >= 2)
        def _():
            wait_flush(half)

        for j in range(16):
            slot = j % DEPTH
            wait_slab(slot)
            r = vec[j]
            l_vec = jnp.full((16,), r & 127, dtype=jnp.int32)

            @pl.when(r < TAIL0)
            def _(slot=slot, l_vec=l_vec, j=j):
                for k in range(4):
                    vals = plsc.load_gather(
                        slabs[slot], [c_iota + (16 * k), l_vec]
                    )
                    off = pl.multiple_of(par + j * D + k * 16, 16)
                    outbuf[pl.ds(off, 16)] = vals

            @pl.when(r >= TAIL0)
            def _(l_vec=l_vec, j=j):
                for k in range(4):
                    vals = plsc.load_gather(tail_buf, [c_iota + (16 * k), l_vec])
                    off = pl.multiple_of(par + j * D + k * 16, 16)
                    outbuf[pl.ds(off, 16)] = vals

            if j < 16 - DEPTH:
                fire(slot, vec[j + DEPTH])
            else:
                @pl.when(c < N_CHUNKS - 1)
                def _(slot=slot, j=j):
                    fire(slot, vec_next[j % DEPTH])
        dst = pl.multiple_of((base + c * 16) * D, 64)
        pltpu.async_copy(
            outbuf.at[pl.ds(par, OUT_W)],
            out_hbm.at[pl.ds(dst, OUT_W)],
            osems[half],
        )

    def chunk(c, carry):
        @pl.when((c & 1) == 0)
        def _():
            chunk_body(c, 0)

        @pl.when((c & 1) == 1)
        def _():
            chunk_body(c, 1)

        return carry

    vec0 = idx_v[pl.ds(0, 16)]
    for j in range(DEPTH):
        fire(j, vec0[j])
    lax.fori_loop(0, N_CHUNKS, chunk, 0)
    wait_flush(0)
    wait_flush(1)


def kernel(param, mesh_indices):
    tail = jnp.pad(param[TAIL0:], ((0, 128 - (V - TAIL0)), (0, 0))).T
    flat = _gather_kernel(param.T, mesh_indices.astype(jnp.int32), tail)
    return flat.reshape(B, D)


# speedup vs baseline: 1.0732x; 1.0732x over previous
import functools

import jax
import jax.numpy as jnp
from jax import lax
from jax.experimental import pallas as pl
from jax.experimental.pallas import tpu as pltpu
from jax.experimental.pallas import tpu_sc as plsc

V = 1000000
TAIL0 = (V // 128) * 128
D = 64
B = 16384
NC = 2
NS = 16
NW = NC * NS
B_PER_W = B // NW
N_CHUNKS = B_PER_W // 16
DEPTH = 8
OUT_W = 16 * D

_mesh = plsc.VectorSubcoreMesh(core_axis_name="c", subcore_axis_name="s")


@functools.partial(
    pl.kernel,
    mesh=_mesh,
    out_type=jax.ShapeDtypeStruct((B * D,), jnp.float32),
    scratch_types=[
        pltpu.VMEM((B_PER_W,), jnp.int32),
        pltpu.VMEM((2 * OUT_W,), jnp.float32),
        pltpu.VMEM((D, 128), jnp.float32),
        *[pltpu.VMEM((D, 128), jnp.float32) for _ in range(DEPTH)],
        *[pltpu.SemaphoreType.DMA for _ in range(DEPTH)],
        pltpu.SemaphoreType.DMA,
        pltpu.SemaphoreType.DMA,
    ],
    compiler_params=pltpu.CompilerParams(needs_layout_passes=False),
)
def _gather_kernel(
    table_hbm, idx_hbm, tail_hbm, out_hbm, idx_v, outbuf, tail_buf, *rest
):
    slabs = rest[:DEPTH]
    sems = rest[DEPTH:2 * DEPTH]
    osems = rest[2 * DEPTH:2 * DEPTH + 2]
    wid = lax.axis_index("s") * NC + lax.axis_index("c")
    base = wid * B_PER_W
    pltpu.sync_copy(idx_hbm.at[pl.ds(base, B_PER_W)], idx_v)
    pltpu.sync_copy(tail_hbm, tail_buf)
    c_iota = lax.iota(jnp.int32, 16)

    def fire(slot, r):
        t128 = pl.multiple_of(
            jnp.where(r < TAIL0, (r >> 7) << 7, 0).astype(jnp.int32), 128
        )
        pltpu.async_copy(table_hbm.at[:, pl.ds(t128, 128)], slabs[slot], sems[slot])

    def wait_slab(slot):
        pltpu.make_async_copy(
            table_hbm.at[:, pl.ds(0, 128)], slabs[slot], sems[slot]
        ).wait()

    def wait_flush(half):
        pltpu.make_async_copy(
            outbuf.at[pl.ds(0, OUT_W)], out_hbm.at[pl.ds(0, OUT_W)], osems[half]
        ).wait()

    def chunk_body(c, half):
        vec = idx_v[pl.ds(c * 16, 16)]
        cnext = jnp.minimum(c + 1, N_CHUNKS - 1)
        vec_next = idx_v[pl.ds(cnext * 16, 16)]
        par = pl.multiple_of(half * OUT_W, 64)
        @pl.when(c >= 2)
        def _():
            wait_flush(half)

        for j in range(16):
            slot = j % DEPTH
            wait_slab(slot)
            r = vec[j]
            l_vec = jnp.full((16,), r & 127, dtype=jnp.int32)

            @pl.when(r < TAIL0)
            def _(slot=slot, l_vec=l_vec, j=j):
                for k in range(4):
                    vals = plsc.load_gather(
                        slabs[slot], [c_iota + (16 * k), l_vec]
                    )
                    off = pl.multiple_of(par + j * D + k * 16, 16)
                    outbuf[pl.ds(off, 16)] = vals

            @pl.when(r >= TAIL0)
            def _(l_vec=l_vec, j=j):
                for k in range(4):
                    vals = plsc.load_gather(tail_buf, [c_iota + (16 * k), l_vec])
                    off = pl.multiple_of(par + j * D + k * 16, 16)
                    outbuf[pl.ds(off, 16)] = vals

            if j < 16 - DEPTH:
                fire(slot, vec[j + DEPTH])
            else:
                @pl.when(c < N_CHUNKS - 1)
                def _(slot=slot, j=j):
                    fire(slot, vec_next[j % DEPTH])
        dst = pl.multiple_of((base + c * 16) * D, 64)
        pltpu.async_copy(
            outbuf.at[pl.ds(par, OUT_W)],
            out_hbm.at[pl.ds(dst, OUT_W)],
            osems[half],
        )

    def chunk(c, carry):
        @pl.when((c & 1) == 0)
        def _():
            chunk_body(c, 0)

        @pl.when((c & 1) == 1)
        def _():
            chunk_body(c, 1)

        return carry

    vec0 = idx_v[pl.ds(0, 16)]
    for j in range(DEPTH):
        fire(j, vec0[j])
    lax.fori_loop(0, N_CHUNKS, chunk, 0)
    wait_flush(0)
    wait_flush(1)


def kernel(param, mesh_indices):
    tail = jnp.pad(param[TAIL0:], ((0, 128 - (V - TAIL0)), (0, 0))).T
    flat = _gather_kernel(param.T, mesh_indices.astype(jnp.int32), tail)
    return flat.reshape(B, D)
